# baseline (speedup 1.0000x reference)
import functools

import jax
import jax.numpy as jnp
from jax import lax
from jax.experimental import pallas as pl
from jax.experimental.pallas import tpu as pltpu
from jax.experimental.pallas import tpu_sc as plsc


def _tc_body(table_ref, w_ref, b_ref, mt_ref):
    mt = jnp.tanh(
        lax.dot_general(
            w_ref[...], table_ref[...],
            (((1,), (1,)), ((), ())),
            preferred_element_type=jnp.float32,
        )
        + b_ref[...].reshape(-1, 1)
    )
    pad = 128 - mt.shape[1]
    mt_ref[...] = jnp.concatenate(
        [mt, jnp.zeros((mt.shape[0], pad), jnp.float32)], axis=1
    )


@functools.lru_cache(maxsize=None)
def _make_sc_gather(B, D):
    info = plsc.get_sparse_core_info()
    nw = info.num_cores * info.num_subcores
    per_w = B // nw
    groups = per_w // 16
    g_per_blk = 8
    mesh = plsc.VectorSubcoreMesh(core_axis_name="c", subcore_axis_name="s")

    @functools.partial(
        pl.kernel,
        mesh=mesh,
        out_type=jax.ShapeDtypeStruct((D, B), jnp.float32),
        scratch_types=[
            pltpu.VMEM((D, 128), jnp.float32),
            pltpu.VMEM((per_w,), jnp.int32),
            pltpu.VMEM((D, per_w), jnp.float32),
            pltpu.SemaphoreType.DMA,
        ],
        compiler_params=pltpu.CompilerParams(needs_layout_passes=False),
    )
    def sc_gather(mt_hbm, idx_hbm, out_hbm, mt_v, idx_v, pout_v, sem):
        wid = lax.axis_index("s") * info.num_cores + lax.axis_index("c")
        base = wid * per_w
        pltpu.sync_copy(mt_hbm, mt_v)
        pltpu.sync_copy(idx_hbm.at[pl.ds(base, per_w)], idx_v)

        out_dmas = []
        for g in range(groups):
            xg = idx_v[pl.ds(g * 16, 16)]
            for c in range(D):
                row = jnp.full((16,), c, jnp.int32)
                pout_v[c, pl.ds(g * 16, 16)] = plsc.load_gather(
                    mt_v, [row, xg]
                )
            if g % g_per_blk == g_per_blk - 1:
                blk = g // g_per_blk
                cols = g_per_blk * 16
                out_dmas.append(
                    pltpu.async_copy(
                        pout_v.at[pl.ds(0, D), pl.ds(blk * cols, cols)],
                        out_hbm.at[pl.ds(0, D), pl.ds(base + blk * cols, cols)],
                        sem,
                    )
                )
        for dma in out_dmas:
            dma.wait()

    return sc_gather


def kernel(x, table, W, b):
    B = x.shape[0]
    D = W.shape[0]
    mt = pl.pallas_call(
        _tc_body,
        out_shape=jax.ShapeDtypeStruct((D, 128), jnp.float32),
    )(table, W, b)
    idx = x.astype(jnp.int32)
    out_t = _make_sc_gather(B, D)(mt, idx)
    return out_t.T


# speedup vs baseline: 1.6126x; 1.6126x over previous
import functools

import jax
import jax.numpy as jnp
from jax import lax
from jax.experimental import pallas as pl
from jax.experimental.pallas import tpu as pltpu
from jax.experimental.pallas import tpu_sc as plsc


def _tc_body(table_ref, w_ref, b_ref, mt_ref):
    mt = jnp.tanh(
        lax.dot_general(
            w_ref[...], table_ref[...],
            (((1,), (1,)), ((), ())),
            preferred_element_type=jnp.float32,
        )
        + b_ref[...].reshape(-1, 1)
    )
    pad = 128 - mt.shape[1]
    mt_ref[...] = jnp.concatenate(
        [mt, jnp.zeros((mt.shape[0], pad), jnp.float32)], axis=1
    )


@functools.lru_cache(maxsize=None)
def _make_sc_gather(B, D):
    info = plsc.get_sparse_core_info()
    nw = info.num_cores * info.num_subcores
    per_w = B // nw
    groups = per_w // 16
    g_per_blk = 8
    mesh = plsc.VectorSubcoreMesh(core_axis_name="c", subcore_axis_name="s")

    @functools.partial(
        pl.kernel,
        mesh=mesh,
        out_type=jax.ShapeDtypeStruct((D, B), jnp.float32),
        scratch_types=[
            pltpu.VMEM((D, 128), jnp.float32),
            pltpu.VMEM((per_w,), jnp.int32),
            pltpu.VMEM((D, per_w), jnp.float32),
            pltpu.SemaphoreType.DMA,
        ],
        compiler_params=pltpu.CompilerParams(needs_layout_passes=False),
    )
    def sc_gather(mt_hbm, idx_hbm, out_hbm, mt_v, idx_v, pout_v, sem):
        wid = lax.axis_index("s") * info.num_cores + lax.axis_index("c")
        base = wid * per_w
        pltpu.sync_copy(mt_hbm, mt_v)
        pltpu.sync_copy(idx_hbm.at[pl.ds(base, per_w)], idx_v)

        @plsc.parallel_loop(0, groups, unroll=2)
        def _gather_body(g):
            xg = idx_v[pl.ds(g * 16, 16)]
            for c in range(D):
                row = jnp.full((16,), c, jnp.int32)
                pout_v[c, pl.ds(g * 16, 16)] = plsc.load_gather(
                    mt_v, [row, xg]
                )

        pltpu.sync_copy(
            pout_v, out_hbm.at[pl.ds(0, D), pl.ds(base, per_w)]
        )

    return sc_gather


def kernel(x, table, W, b):
    B = x.shape[0]
    D = W.shape[0]
    mt = pl.pallas_call(
        _tc_body,
        out_shape=jax.ShapeDtypeStruct((D, 128), jnp.float32),
    )(table, W, b)
    idx = x.astype(jnp.int32)
    out_t = _make_sc_gather(B, D)(mt, idx)
    return out_t.T


# speedup vs baseline: 1.6208x; 1.0051x over previous
import functools

import jax
import jax.numpy as jnp
from jax import lax
from jax.experimental import pallas as pl
from jax.experimental.pallas import tpu as pltpu
from jax.experimental.pallas import tpu_sc as plsc


def _tc_body(table_ref, w_ref, b_ref, mt_ref):
    mt = jnp.tanh(
        lax.dot_general(
            w_ref[...], table_ref[...],
            (((1,), (1,)), ((), ())),
            preferred_element_type=jnp.float32,
        )
        + b_ref[...].reshape(-1, 1)
    )
    pad = 128 - mt.shape[1]
    mt_ref[...] = jnp.concatenate(
        [mt, jnp.zeros((mt.shape[0], pad), jnp.float32)], axis=1
    )


@functools.lru_cache(maxsize=None)
def _make_sc_gather(B, D):
    info = plsc.get_sparse_core_info()
    nw = info.num_cores * info.num_subcores
    per_w = B // nw
    groups = per_w // 16
    g_per_blk = 8
    mesh = plsc.VectorSubcoreMesh(core_axis_name="c", subcore_axis_name="s")

    @functools.partial(
        pl.kernel,
        mesh=mesh,
        out_type=jax.ShapeDtypeStruct((D, B), jnp.float32),
        scratch_types=[
            pltpu.VMEM((D, 128), jnp.float32),
            pltpu.VMEM((per_w,), jnp.int32),
            pltpu.VMEM((D, per_w), jnp.float32),
            pltpu.SemaphoreType.DMA,
        ],
        compiler_params=pltpu.CompilerParams(
            needs_layout_passes=False, skip_device_barrier=True
        ),
    )
    def sc_gather(mt_hbm, idx_hbm, out_hbm, mt_v, idx_v, pout_v, sem):
        wid = lax.axis_index("s") * info.num_cores + lax.axis_index("c")
        base = wid * per_w
        pltpu.sync_copy(mt_hbm, mt_v)
        pltpu.sync_copy(idx_hbm.at[pl.ds(base, per_w)], idx_v)

        @plsc.parallel_loop(0, groups, unroll=2)
        def _gather_body(g):
            xg = idx_v[pl.ds(g * 16, 16)]
            for c in range(D):
                row = jnp.full((16,), c, jnp.int32)
                pout_v[c, pl.ds(g * 16, 16)] = plsc.load_gather(
                    mt_v, [row, xg]
                )

        pltpu.sync_copy(
            pout_v, out_hbm.at[pl.ds(0, D), pl.ds(base, per_w)]
        )

    return sc_gather


def kernel(x, table, W, b):
    B = x.shape[0]
    D = W.shape[0]
    mt = pl.pallas_call(
        _tc_body,
        out_shape=jax.ShapeDtypeStruct((D, 128), jnp.float32),
    )(table, W, b)
    idx = x.astype(jnp.int32)
    out_t = _make_sc_gather(B, D)(mt, idx)
    return out_t.T
